# baseline (speedup 1.0000x reference)
import jax
import jax.numpy as jnp
from jax.experimental import pallas as pl
from jax.experimental.pallas import tpu as pltpu
from jax.experimental.pallas import tpu_sc as plsc

_B = 32768
_K = 8192
_D = 256
_CC = 0.25

_C = 4
_CB = _B // _C
_BB = 256
_NBC = _CB // _BB
_KC = 2048
_NKC = _K // _KC
_EB = 2048
_NEC = _CB // _EB
_GW = 128


def _esq_body(e_ref, esq_ref, ebf_ref):
    e = e_ref[...]
    esq_ref[...] = jnp.sum(e * e, axis=1)[None, :]
    ebf_ref[...] = e.astype(jnp.bfloat16)


def _esq(embedding):
    return pl.pallas_call(
        _esq_body,
        out_shape=[
            jax.ShapeDtypeStruct((1, _K), jnp.float32),
            jax.ShapeDtypeStruct((_K, _D), jnp.bfloat16),
        ],
    )(embedding)


def _argmin_body(z_ref, ebf_ref, esq_ref, idx_ref, iota_ref):
    b = pl.program_id(0)

    @pl.when(b == 0)
    def _():
        iota_ref[...] = jax.lax.broadcasted_iota(jnp.int32, (_BB, _K), 1)

    z = z_ref[...]
    zsq = jnp.sum(z * z, axis=1, keepdims=True)

    z2 = (z + z).astype(jnp.bfloat16)
    mm2 = jax.lax.dot_general(z2, ebf_ref[...], (((1,), (1,)), ((), ())),
                              preferred_element_type=jnp.float32)
    dist = (zsq + esq_ref[...]) - mm2
    t = dist - zsq
    key = (t * 536870912.0).astype(jnp.int32) + iota_ref[...]
    kmin = jnp.min(key, axis=1)
    idx_ref[0, 0, :] = jnp.bitwise_and(kmin, _K - 1)


def _encode_chunk(z, ebf, esq, c):
    idx3 = pl.pallas_call(
        _argmin_body,
        grid=(_NBC,),
        in_specs=[
            pl.BlockSpec((_BB, _D), lambda b, c=c: (c * _NBC + b, 0)),
            pl.BlockSpec((_K, _D), lambda b: (0, 0)),
            pl.BlockSpec((1, _K), lambda b: (0, 0)),
        ],
        out_specs=pl.BlockSpec((1, 1, _BB), lambda b: (b, 0, 0)),
        out_shape=jax.ShapeDtypeStruct((_NBC, 1, _BB), jnp.int32),
        scratch_shapes=[pltpu.VMEM((_BB, _K), jnp.int32)],
    )(z, ebf, esq)
    return idx3.reshape(_CB)


def _gather_chunk(embedding, indices_chunk):
    idx2 = indices_chunk.reshape(1, _CB)

    @pl.kernel(
        out_type=jax.ShapeDtypeStruct((_CB, _D), jnp.float32),
        mesh=plsc.VectorSubcoreMesh(core_axis_name="c", subcore_axis_name="s"),
    )
    def k(emb_hbm, i_hbm, o_hbm):
        def body(i_vmem, o_vmem):
            pltpu.sync_copy(emb_hbm.at[i_vmem.at[0]], o_vmem)

        pltpu.emit_pipeline(
            body,
            grid=(_CB // _GW,),
            in_specs=[pl.BlockSpec((1, _GW), lambda i: (0, i))],
            out_specs=[pl.BlockSpec((_GW, _D), lambda i: (i, 0))],
            core_axis_name=("c", "s"),
            dimension_semantics=(pltpu.PARALLEL,),
        )(i_hbm, o_hbm)

    return k(embedding, idx2)


def _st_loss_core(z_ref, zq_ref, zst_ref, loss_ref, acc_ref):
    b = pl.program_id(0)

    @pl.when(b == 0)
    def _():
        acc_ref[0, 0] = 0.0

    z = z_ref[...]
    d = zq_ref[...] - z
    zst_ref[...] = z + d
    acc_ref[0, 0] += jnp.sum(d * d)

    @pl.when(b == _NEC - 1)
    def _():
        m = acc_ref[0, 0] / (_B * _D)
        loss_ref[0, 0] = m + _CC * m


def _st_loss_body(z_ref, zq_ref, buf_ref, zst_ref, loss_ref, acc_ref):
    _st_loss_core(z_ref, zq_ref, zst_ref, loss_ref, acc_ref)


def _st_loss_chunk(z, z_q_chunk, zst_buf, c):
    out_specs = [
        pl.BlockSpec((_EB, _D), lambda b, c=c: (c * _NEC + b, 0)),
        pl.BlockSpec(memory_space=pltpu.SMEM),
    ]
    out_shape = [
        jax.ShapeDtypeStruct((_B, _D), jnp.float32),
        jax.ShapeDtypeStruct((1, 1), jnp.float32),
    ]
    z_spec = pl.BlockSpec((_EB, _D), lambda b, c=c: (c * _NEC + b, 0))
    zq_spec = pl.BlockSpec((_EB, _D), lambda b: (b, 0))
    if zst_buf is None:
        z_q_st, loss = pl.pallas_call(
            _st_loss_core,
            grid=(_NEC,),
            in_specs=[z_spec, zq_spec],
            out_specs=out_specs,
            out_shape=out_shape,
            scratch_shapes=[pltpu.SMEM((1, 1), jnp.float32)],
        )(z, z_q_chunk)
    else:
        z_q_st, loss = pl.pallas_call(
            _st_loss_body,
            grid=(_NEC,),
            in_specs=[z_spec, zq_spec, pl.BlockSpec(memory_space=pl.ANY)],
            out_specs=out_specs,
            out_shape=out_shape,
            scratch_shapes=[pltpu.SMEM((1, 1), jnp.float32)],
            input_output_aliases={2: 0},
        )(z, z_q_chunk, zst_buf)
    return z_q_st, loss[0, 0]


def kernel(z, embedding):
    esq, ebf = _esq(embedding)
    idx_chunks = [_encode_chunk(z, ebf, esq, c) for c in range(_C)]
    zq_chunks = [_gather_chunk(embedding, idx_chunks[c]) for c in range(_C)]
    zst_buf = None
    loss = None
    for c in range(_C):
        zst_buf, part = _st_loss_chunk(z, zq_chunks[c], zst_buf, c)
        loss = part if loss is None else loss + part
    indices = jnp.concatenate(idx_chunks, axis=0)
    return (zst_buf, loss, indices)


# speedup vs baseline: 1.2663x; 1.2663x over previous
import jax
import jax.numpy as jnp
from jax.experimental import pallas as pl
from jax.experimental.pallas import tpu as pltpu
from jax.experimental.pallas import tpu_sc as plsc

_B = 32768
_K = 8192
_D = 256
_CC = 0.25

_C = 4
_CB = _B // _C
_BB = 256
_NBC = _CB // _BB
_EB = 2048
_NEC = _CB // _EB
_GW = 128


def _prep_body(e_ref, esq_ref, ebf_ref):
    e = e_ref[...]
    esq_ref[...] = jnp.sum(e * e, axis=1)[None, :]
    ebf_ref[...] = e.astype(jnp.bfloat16)


def _prep(embedding):
    return pl.pallas_call(
        _prep_body,
        out_shape=[
            jax.ShapeDtypeStruct((1, _K), jnp.float32),
            jax.ShapeDtypeStruct((_K, _D), jnp.bfloat16),
        ],
    )(embedding)


def _argmin_body(z_ref, ebf_ref, esq_ref, idx_ref):
    z = z_ref[...]
    zsq = jnp.sum(z * z, axis=1, keepdims=True)

    mm = jax.lax.dot_general(z.astype(jnp.bfloat16), ebf_ref[...],
                             (((1,), (1,)), ((), ())),
                             preferred_element_type=jnp.float32)
    dist = (zsq + esq_ref[...]) - 2.0 * mm
    lmin = jnp.min(dist, axis=1, keepdims=True)
    iota = jax.lax.broadcasted_iota(jnp.int32, dist.shape, 1)
    idx_ref[0, 0, :] = jnp.min(jnp.where(dist == lmin, iota, _K), axis=1)


def _encode_chunk(z, ebf, esq, c):
    idx3 = pl.pallas_call(
        _argmin_body,
        grid=(_NBC,),
        in_specs=[
            pl.BlockSpec((_BB, _D), lambda b, c=c: (c * _NBC + b, 0)),
            pl.BlockSpec((_K, _D), lambda b: (0, 0)),
            pl.BlockSpec((1, _K), lambda b: (0, 0)),
        ],
        out_specs=pl.BlockSpec((1, 1, _BB), lambda b: (b, 0, 0)),
        out_shape=jax.ShapeDtypeStruct((_NBC, 1, _BB), jnp.int32),
    )(z, ebf, esq)
    return idx3.reshape(_CB)


def _gather_chunk(embedding, indices_chunk):
    idx2 = indices_chunk.reshape(1, _CB)

    @pl.kernel(
        out_type=jax.ShapeDtypeStruct((_CB, _D), jnp.float32),
        mesh=plsc.VectorSubcoreMesh(core_axis_name="c", subcore_axis_name="s"),
    )
    def k(emb_hbm, i_hbm, o_hbm):
        def body(i_vmem, o_vmem):
            pltpu.sync_copy(emb_hbm.at[i_vmem.at[0]], o_vmem)

        pltpu.emit_pipeline(
            body,
            grid=(_CB // _GW,),
            in_specs=[pl.BlockSpec((1, _GW), lambda i: (0, i))],
            out_specs=[pl.BlockSpec((_GW, _D), lambda i: (i, 0))],
            core_axis_name=("c", "s"),
            dimension_semantics=(pltpu.PARALLEL,),
        )(i_hbm, o_hbm)

    return k(embedding, idx2)


def _st_loss_core(z_ref, zq_ref, zst_ref, loss_ref, acc_ref):
    b = pl.program_id(0)

    @pl.when(b == 0)
    def _():
        acc_ref[0, 0] = 0.0

    z = z_ref[...]
    d = zq_ref[...] - z
    zst_ref[...] = z + d
    acc_ref[0, 0] += jnp.sum(d * d)

    @pl.when(b == _NEC - 1)
    def _():
        m = acc_ref[0, 0] / (_B * _D)
        loss_ref[0, 0] = m + _CC * m


def _st_loss_body(z_ref, zq_ref, buf_ref, zst_ref, loss_ref, acc_ref):
    _st_loss_core(z_ref, zq_ref, zst_ref, loss_ref, acc_ref)


def _st_loss_chunk(z, z_q_chunk, zst_buf, c):
    out_specs = [
        pl.BlockSpec((_EB, _D), lambda b, c=c: (c * _NEC + b, 0)),
        pl.BlockSpec(memory_space=pltpu.SMEM),
    ]
    out_shape = [
        jax.ShapeDtypeStruct((_B, _D), jnp.float32),
        jax.ShapeDtypeStruct((1, 1), jnp.float32),
    ]
    z_spec = pl.BlockSpec((_EB, _D), lambda b, c=c: (c * _NEC + b, 0))
    zq_spec = pl.BlockSpec((_EB, _D), lambda b: (b, 0))
    if zst_buf is None:
        z_q_st, loss = pl.pallas_call(
            _st_loss_core,
            grid=(_NEC,),
            in_specs=[z_spec, zq_spec],
            out_specs=out_specs,
            out_shape=out_shape,
            scratch_shapes=[pltpu.SMEM((1, 1), jnp.float32)],
        )(z, z_q_chunk)
    else:
        z_q_st, loss = pl.pallas_call(
            _st_loss_body,
            grid=(_NEC,),
            in_specs=[z_spec, zq_spec, pl.BlockSpec(memory_space=pl.ANY)],
            out_specs=out_specs,
            out_shape=out_shape,
            scratch_shapes=[pltpu.SMEM((1, 1), jnp.float32)],
            input_output_aliases={2: 0},
        )(z, z_q_chunk, zst_buf)
    return z_q_st, loss[0, 0]


def kernel(z, embedding):
    esq, ebf = _prep(embedding)
    idx_chunks = [_encode_chunk(z, ebf, esq, c) for c in range(_C)]
    zq_chunks = [_gather_chunk(embedding, idx_chunks[c]) for c in range(_C)]
    zst_buf = None
    loss = None
    for c in range(_C):
        zst_buf, part = _st_loss_chunk(z, zq_chunks[c], zst_buf, c)
        loss = part if loss is None else loss + part
    indices = jnp.concatenate(idx_chunks, axis=0)
    return (zst_buf, loss, indices)
